# baseline (speedup 1.0000x reference)
import jax
import jax.numpy as jnp
from jax.experimental import pallas as pl
from jax.experimental.pallas import tpu as pltpu

N_PROTOS = 8192
BQ = 512


def _body(q_ref, p_ref, out_ref, pn_ref):
    @pl.when(pl.program_id(0) == 0)
    def _init():
        p = p_ref[...]
        ones = jnp.ones((1, p.shape[1]), jnp.float32)
        pn_ref[...] = jax.lax.dot_general(
            ones, p * p, (((1,), (1,)), ((), ())),
            preferred_element_type=jnp.float32,
        )

    q = q_ref[0]
    s = jax.lax.dot_general(
        q, p_ref[...], (((1,), (1,)), ((), ())),
        preferred_element_type=jnp.float32,
    )
    m = jnp.max(s, axis=1, keepdims=True)
    pn_sel = jnp.min(
        jnp.where(s == m, pn_ref[...], jnp.float32(jnp.inf)), axis=1
    )
    qn = jnp.sum(q * q, axis=1)
    out_ref[0, 0, :] = qn - 2.0 * m[:, 0] + pn_sel


@jax.jit
def kernel(queries, prototypes):
    B, L, C = queries.shape
    n_lb = L // BQ
    grid = (B * n_lb,)
    out = pl.pallas_call(
        _body,
        grid=grid,
        in_specs=[
            pl.BlockSpec((1, BQ, C), lambda g: (g // n_lb, g % n_lb, 0)),
            pl.BlockSpec(prototypes.shape, lambda g: (0, 0)),
        ],
        out_specs=pl.BlockSpec((1, 1, BQ), lambda g: (g, 0, 0)),
        out_shape=jax.ShapeDtypeStruct((B * n_lb, 1, BQ), jnp.float32),
        scratch_shapes=[pltpu.VMEM((1, N_PROTOS), jnp.float32)],
    )(queries, prototypes)
    return out.reshape(B, L)


# speedup vs baseline: 1.0477x; 1.0477x over previous
import jax
import jax.numpy as jnp
from jax.experimental import pallas as pl
from jax.experimental.pallas import tpu as pltpu

N_PROTOS = 8192
NQ = 8192
CHUNK = 512
NCHUNK = 4
BQ = CHUNK * NCHUNK


def _body(q_ref, p_ref, out_ref, pn_ref):
    p = p_ref[...]

    @pl.when(pl.program_id(0) == 0)
    def _init():
        ones = jnp.ones((1, p.shape[1]), jnp.float32)
        pn_ref[...] = jax.lax.dot_general(
            ones, p * p, (((1,), (1,)), ((), ())),
            preferred_element_type=jnp.float32,
        )

    for c in range(NCHUNK):
        q = q_ref[pl.ds(c * CHUNK, CHUNK), :]
        s = jax.lax.dot_general(
            q, p, (((1,), (1,)), ((), ())),
            preferred_element_type=jnp.float32,
        )
        m = jnp.max(s, axis=1, keepdims=True)
        pn_sel = jnp.min(
            jnp.where(s == m, pn_ref[...], jnp.float32(jnp.inf)), axis=1
        )
        qn = jnp.sum(q * q, axis=1)
        out_ref[0, 0, pl.ds(c * CHUNK, CHUNK)] = qn - 2.0 * m[:, 0] + pn_sel


@jax.jit
def kernel(queries, prototypes):
    B, L, C = queries.shape
    grid = (NQ // BQ,)
    out = pl.pallas_call(
        _body,
        grid=grid,
        in_specs=[
            pl.BlockSpec((BQ, C), lambda g: (g, 0)),
            pl.BlockSpec(prototypes.shape, lambda g: (0, 0)),
        ],
        out_specs=pl.BlockSpec((1, 1, BQ), lambda g: (g, 0, 0)),
        out_shape=jax.ShapeDtypeStruct((NQ // BQ, 1, BQ), jnp.float32),
        scratch_shapes=[pltpu.VMEM((1, N_PROTOS), jnp.float32)],
    )(queries.reshape(NQ, C), prototypes)
    return out.reshape(B, L)
